# baseline (speedup 1.0000x reference)
import functools

import jax
import jax.numpy as jnp
from jax import lax
from jax.experimental import pallas as pl
from jax.experimental.pallas import tpu as pltpu
from jax.experimental.pallas import tpu_sc as plsc

_NC = 2
_NS = 16
_NW = _NC * _NS

_CHUNK = 2048


def _sc_gather_combine(idx, x, intercepts, slopes):
  n = idx.shape[0]
  per_w = n // _NW
  chunks = per_w // _CHUNK
  mesh = plsc.VectorSubcoreMesh(core_axis_name="c", subcore_axis_name="s")

  @functools.partial(
      pl.kernel,
      mesh=mesh,
      out_type=jax.ShapeDtypeStruct((n,), jnp.float32),
      scratch_types=[
          pltpu.VMEM((_CHUNK,), jnp.int32),
          pltpu.VMEM((_CHUNK,), jnp.float32),
          pltpu.VMEM((_CHUNK,), jnp.float32),
          pltpu.VMEM((_CHUNK,), jnp.float32),
          pltpu.VMEM((_CHUNK,), jnp.float32),
          pltpu.SemaphoreType.DMA,
          pltpu.SemaphoreType.DMA,
      ],
  )
  def run(idx_hbm, x_hbm, ri_hbm, rs_hbm, out_hbm,
          idx_v, x_v, ri_v, rs_v, g_v, sem_a, sem_b):
    wid = lax.axis_index("s") * _NC + lax.axis_index("c")
    base0 = wid * per_w

    def chunk_body(ci, carry):
      base = base0 + ci * _CHUNK
      pltpu.sync_copy(idx_hbm.at[pl.ds(base, _CHUNK)], idx_v)
      cp_ri = pltpu.async_copy(ri_hbm.at[idx_v], ri_v, sem_a)
      cp_rs = pltpu.async_copy(rs_hbm.at[idx_v], rs_v, sem_b)
      pltpu.sync_copy(x_hbm.at[pl.ds(base, _CHUNK)], x_v)
      cp_ri.wait()
      cp_rs.wait()

      def vec_body(j, c2):
        sl = pl.ds(j * 16, 16)
        g_v[sl] = ri_v[sl] + x_v[sl] * rs_v[sl]
        return c2

      lax.fori_loop(0, _CHUNK // 16, vec_body, 0)
      pltpu.sync_copy(g_v, out_hbm.at[pl.ds(base, _CHUNK)])
      return carry

    lax.fori_loop(0, chunks, chunk_body, 0)

  return run(idx, x, intercepts, slopes)


_ROWS = 8192


def _tc_matvec_add(xf, w, g):
  n, p = xf.shape

  def body(x_ref, w_ref, g_ref, o_ref):
    fe = jax.lax.dot_general(
        x_ref[...], w_ref[...], (((1,), (0,)), ((), ())),
        preferred_element_type=jnp.float32)
    o_ref[...] = fe + g_ref[...]

  return pl.pallas_call(
      body,
      grid=(n // _ROWS,),
      in_specs=[
          pl.BlockSpec((_ROWS, p), lambda i: (i, 0)),
          pl.BlockSpec((p, 1), lambda i: (0, 0)),
          pl.BlockSpec((_ROWS, 1), lambda i: (i, 0)),
      ],
      out_specs=pl.BlockSpec((_ROWS, 1), lambda i: (i, 0)),
      out_shape=jax.ShapeDtypeStruct((n, 1), jnp.float32),
  )(xf, w.reshape(p, 1), g.reshape(n, 1))


def kernel(X_fixed, subject_indices, X_random_slope, fixed_effects,
           random_intercepts, random_slopes):
  n = X_fixed.shape[0]
  p = X_fixed.shape[-1]
  xf = X_fixed.reshape(n, p)
  idx = subject_indices.reshape(n)
  x = X_random_slope.reshape(n)
  g = _sc_gather_combine(idx, x, random_intercepts, random_slopes)
  return _tc_matvec_add(xf, fixed_effects, g)


# speedup vs baseline: 6.4563x; 6.4563x over previous
import functools

import jax
import jax.numpy as jnp
from jax import lax
from jax.experimental import pallas as pl
from jax.experimental.pallas import tpu as pltpu
from jax.experimental.pallas import tpu_sc as plsc

_NC = 2
_NS = 16
_NW = _NC * _NS

_CHUNK = 2048


def _sc_gather_combine(idx, x, intercepts, slopes):
  n = idx.shape[0]
  per_w = n // _NW
  chunks = per_w // _CHUNK
  mesh = plsc.VectorSubcoreMesh(core_axis_name="c", subcore_axis_name="s")

  @functools.partial(
      pl.kernel,
      mesh=mesh,
      out_type=jax.ShapeDtypeStruct((n,), jnp.float32),
      scratch_types=[
          pltpu.VMEM((_CHUNK,), jnp.int32),
          pltpu.VMEM((_CHUNK,), jnp.float32),
          pltpu.VMEM((_CHUNK,), jnp.float32),
          pltpu.VMEM((_CHUNK,), jnp.float32),
          pltpu.VMEM((_CHUNK,), jnp.float32),
          pltpu.SemaphoreType.DMA,
          pltpu.SemaphoreType.DMA,
      ],
  )
  def run(idx_hbm, x_hbm, ri_hbm, rs_hbm, out_hbm,
          idx_v, x_v, ri_v, rs_v, g_v, sem_a, sem_b):
    wid = lax.axis_index("s") * _NC + lax.axis_index("c")
    base0 = wid * per_w

    def chunk_body(ci, carry):
      base = base0 + ci * _CHUNK
      pltpu.sync_copy(idx_hbm.at[pl.ds(base, _CHUNK)], idx_v)
      cp_ri = pltpu.async_copy(ri_hbm.at[idx_v], ri_v, sem_a)
      cp_rs = pltpu.async_copy(rs_hbm.at[idx_v], rs_v, sem_b)
      pltpu.sync_copy(x_hbm.at[pl.ds(base, _CHUNK)], x_v)
      cp_ri.wait()
      cp_rs.wait()

      def vec_body(j, c2):
        sl = pl.ds(j * 16, 16)
        g_v[sl] = ri_v[sl] + x_v[sl] * rs_v[sl]
        return c2

      lax.fori_loop(0, _CHUNK // 16, vec_body, 0)
      pltpu.sync_copy(g_v, out_hbm.at[pl.ds(base, _CHUNK)])
      return carry

    lax.fori_loop(0, chunks, chunk_body, 0)

  return run(idx, x, intercepts, slopes)


_COLS = 32768


def _tc_matvec_add(xT, w, g):
  p, n = xT.shape

  def body(w_ref, x_ref, g_ref, o_ref):
    fe = jax.lax.dot_general(
        w_ref[...], x_ref[...], (((1,), (0,)), ((), ())),
        preferred_element_type=jnp.float32)
    o_ref[...] = fe + g_ref[...]

  return pl.pallas_call(
      body,
      grid=(n // _COLS,),
      in_specs=[
          pl.BlockSpec((1, p), lambda i: (0, 0)),
          pl.BlockSpec((p, _COLS), lambda i: (0, i)),
          pl.BlockSpec((1, _COLS), lambda i: (0, i)),
      ],
      out_specs=pl.BlockSpec((1, _COLS), lambda i: (0, i)),
      out_shape=jax.ShapeDtypeStruct((1, n), jnp.float32),
  )(w.reshape(1, p), xT, g.reshape(1, n))


def kernel(X_fixed, subject_indices, X_random_slope, fixed_effects,
           random_intercepts, random_slopes):
  n, _, p = X_fixed.shape
  xT = jnp.transpose(X_fixed, (2, 1, 0)).reshape(p, n)
  idx = subject_indices.reshape(n)
  x = X_random_slope.reshape(n)
  g = _sc_gather_combine(idx, x, random_intercepts, random_slopes)
  out = _tc_matvec_add(xT, fixed_effects, g)
  return out.reshape(n, 1)


# speedup vs baseline: 7.7056x; 1.1935x over previous
import functools

import jax
import jax.numpy as jnp
from jax import lax
from jax.experimental import pallas as pl
from jax.experimental.pallas import tpu as pltpu
from jax.experimental.pallas import tpu_sc as plsc

_NC = 2
_NS = 16
_NW = _NC * _NS

_CHUNK = 2048


def _sc_gather_combine(idx, x, intercepts, slopes):
  n = idx.shape[0]
  per_w = n // _NW
  chunks = per_w // _CHUNK
  mesh = plsc.VectorSubcoreMesh(core_axis_name="c", subcore_axis_name="s")

  @functools.partial(
      pl.kernel,
      mesh=mesh,
      out_type=jax.ShapeDtypeStruct((n,), jnp.float32),
      scratch_types=[
          pltpu.VMEM((_CHUNK,), jnp.int32),
          pltpu.VMEM((_CHUNK,), jnp.float32),
          pltpu.VMEM((_CHUNK,), jnp.float32),
          pltpu.VMEM((_CHUNK,), jnp.float32),
          pltpu.VMEM((_CHUNK,), jnp.float32),
          pltpu.SemaphoreType.DMA,
          pltpu.SemaphoreType.DMA,
      ],
  )
  def run(idx_hbm, x_hbm, ri_hbm, rs_hbm, out_hbm,
          idx_v, x_v, ri_v, rs_v, g_v, sem_a, sem_b):
    wid = lax.axis_index("s") * _NC + lax.axis_index("c")
    base0 = wid * per_w

    def chunk_body(ci, carry):
      base = base0 + ci * _CHUNK
      pltpu.sync_copy(idx_hbm.at[pl.ds(base, _CHUNK)], idx_v)
      cp_ri = pltpu.async_copy(ri_hbm.at[idx_v], ri_v, sem_a)
      cp_rs = pltpu.async_copy(rs_hbm.at[idx_v], rs_v, sem_b)
      pltpu.sync_copy(x_hbm.at[pl.ds(base, _CHUNK)], x_v)
      cp_ri.wait()
      cp_rs.wait()

      def vec_body(j, c2):
        sl = pl.ds(j * 16, 16)
        g_v[sl] = ri_v[sl] + x_v[sl] * rs_v[sl]
        return c2

      lax.fori_loop(0, _CHUNK // 16, vec_body, 0)
      pltpu.sync_copy(g_v, out_hbm.at[pl.ds(base, _CHUNK)])
      return carry

    lax.fori_loop(0, chunks, chunk_body, 0)

  return run(idx, x, intercepts, slopes)


_COLS = 32768


def _tc_matvec(xT, w):
  p, n = xT.shape

  def body(w_ref, x_ref, o_ref):
    o_ref[...] = jax.lax.dot_general(
        w_ref[...], x_ref[...], (((1,), (0,)), ((), ())),
        preferred_element_type=jnp.float32)

  return pl.pallas_call(
      body,
      grid=(n // _COLS,),
      in_specs=[
          pl.BlockSpec((1, p), lambda i: (0, 0)),
          pl.BlockSpec((p, _COLS), lambda i: (0, i)),
      ],
      out_specs=pl.BlockSpec((1, _COLS), lambda i: (0, i)),
      out_shape=jax.ShapeDtypeStruct((1, n), jnp.float32),
  )(w.reshape(1, p), xT)


_CCOLS = 131072


def _tc_combine(fe, g):
  _, n = fe.shape

  def body(a_ref, b_ref, o_ref):
    o_ref[...] = a_ref[...] + b_ref[...]

  return pl.pallas_call(
      body,
      grid=(n // _CCOLS,),
      in_specs=[
          pl.BlockSpec((1, _CCOLS), lambda i: (0, i)),
          pl.BlockSpec((1, _CCOLS), lambda i: (0, i)),
      ],
      out_specs=pl.BlockSpec((1, _CCOLS), lambda i: (0, i)),
      out_shape=jax.ShapeDtypeStruct((1, n), jnp.float32),
  )(fe, g)


def kernel(X_fixed, subject_indices, X_random_slope, fixed_effects,
           random_intercepts, random_slopes):
  n, _, p = X_fixed.shape
  xT = jnp.transpose(X_fixed, (2, 1, 0)).reshape(p, n)
  idx = subject_indices.reshape(n)
  x = X_random_slope.reshape(n)
  g = _sc_gather_combine(idx, x, random_intercepts, random_slopes)
  fe = _tc_matvec(xT, fixed_effects)
  out = _tc_combine(fe, g.reshape(1, n))
  return out.reshape(n, 1)


# speedup vs baseline: 10.8268x; 1.4050x over previous
import functools

import jax
import jax.numpy as jnp
from jax import lax
from jax.experimental import pallas as pl
from jax.experimental.pallas import tpu as pltpu
from jax.experimental.pallas import tpu_sc as plsc

_NC = 2
_NS = 16
_NW = _NC * _NS

_CHUNK = 2048
_SEG = 25000


def _sc_gather_combine(idx, x, packed):
  n = idx.shape[0]
  s = packed.shape[0]
  per_w = n // _NW
  chunks = per_w // _CHUNK
  n_pieces = s // _SEG
  stage_iters = (n_pieces + _NS - 1) // _NS
  mesh = plsc.VectorSubcoreMesh(core_axis_name="c", subcore_axis_name="s")

  @functools.partial(
      pl.kernel,
      mesh=mesh,
      out_type=jax.ShapeDtypeStruct((n,), jnp.float32),
      scratch_types=[
          pltpu.VMEM_SHARED((s,), jnp.int32),
          pltpu.VMEM((_SEG,), jnp.int32),
          pltpu.VMEM((_CHUNK,), jnp.int32),
          pltpu.VMEM((_CHUNK,), jnp.int32),
          pltpu.VMEM((_CHUNK,), jnp.float32),
          pltpu.VMEM((_CHUNK,), jnp.float32),
          pltpu.SemaphoreType.DMA,
      ],
  )
  def run(idx_hbm, x_hbm, tab_hbm, out_hbm,
          s_tab, stage_v, idx_v, p_v, x_v, g_v, sem):
    sid = lax.axis_index("s")
    wid = sid * _NC + lax.axis_index("c")
    base0 = wid * per_w

    def stage_body(k, carry):
      piece = k * _NS + sid

      @pl.when(piece < n_pieces)
      def _():
        off = piece * _SEG
        pltpu.sync_copy(tab_hbm.at[pl.ds(off, _SEG)], stage_v)
        pltpu.sync_copy(stage_v, s_tab.at[pl.ds(off, _SEG)])

      return carry

    lax.fori_loop(0, stage_iters, stage_body, 0)
    plsc.subcore_barrier()

    mask = jnp.int32(-65536)

    def chunk_body(ci, carry):
      base = base0 + ci * _CHUNK
      pltpu.sync_copy(idx_hbm.at[pl.ds(base, _CHUNK)], idx_v)
      cp = pltpu.async_copy(s_tab.at[idx_v], p_v, sem)
      pltpu.sync_copy(x_hbm.at[pl.ds(base, _CHUNK)], x_v)
      cp.wait()

      def vec_body(j, c2):
        sl = pl.ds(j * 16, 16)
        v = p_v[sl]
        ri = jax.lax.bitcast_convert_type(v << 16, jnp.float32)
        rs = jax.lax.bitcast_convert_type(v & mask, jnp.float32)
        g_v[sl] = ri + x_v[sl] * rs
        return c2

      lax.fori_loop(0, _CHUNK // 16, vec_body, 0)
      pltpu.sync_copy(g_v, out_hbm.at[pl.ds(base, _CHUNK)])
      return carry

    lax.fori_loop(0, chunks, chunk_body, 0)

  return run(idx, x, packed)


_COLS = 32768


def _tc_matvec(xT, w):
  p, n = xT.shape

  def body(w_ref, x_ref, o_ref):
    o_ref[...] = jax.lax.dot_general(
        w_ref[...], x_ref[...], (((1,), (0,)), ((), ())),
        preferred_element_type=jnp.float32)

  return pl.pallas_call(
      body,
      grid=(n // _COLS,),
      in_specs=[
          pl.BlockSpec((1, p), lambda i: (0, 0)),
          pl.BlockSpec((p, _COLS), lambda i: (0, i)),
      ],
      out_specs=pl.BlockSpec((1, _COLS), lambda i: (0, i)),
      out_shape=jax.ShapeDtypeStruct((1, n), jnp.float32),
  )(w.reshape(1, p), xT)


_CCOLS = 131072


def _tc_combine(fe, g):
  _, n = fe.shape

  def body(a_ref, b_ref, o_ref):
    o_ref[...] = a_ref[...] + b_ref[...]

  return pl.pallas_call(
      body,
      grid=(n // _CCOLS,),
      in_specs=[
          pl.BlockSpec((1, _CCOLS), lambda i: (0, i)),
          pl.BlockSpec((1, _CCOLS), lambda i: (0, i)),
      ],
      out_specs=pl.BlockSpec((1, _CCOLS), lambda i: (0, i)),
      out_shape=jax.ShapeDtypeStruct((1, n), jnp.float32),
  )(fe, g)


def kernel(X_fixed, subject_indices, X_random_slope, fixed_effects,
           random_intercepts, random_slopes):
  n, _, p = X_fixed.shape
  xT = jnp.transpose(X_fixed, (2, 1, 0)).reshape(p, n)
  idx = subject_indices.reshape(n)
  x = X_random_slope.reshape(n)
  ri_u = jax.lax.bitcast_convert_type(
      random_intercepts.astype(jnp.bfloat16), jnp.uint16).astype(jnp.uint32)
  rs_u = jax.lax.bitcast_convert_type(
      random_slopes.astype(jnp.bfloat16), jnp.uint16).astype(jnp.uint32)
  packed = jax.lax.bitcast_convert_type(ri_u | (rs_u << 16), jnp.int32)
  g = _sc_gather_combine(idx, x, packed)
  fe = _tc_matvec(xT, fixed_effects)
  out = _tc_combine(fe, g.reshape(1, n))
  return out.reshape(n, 1)
